# baseline (speedup 1.0000x reference)
import functools

import jax
import jax.numpy as jnp
from jax import lax
from jax.experimental import pallas as pl
from jax.experimental.pallas import tpu as pltpu
from jax.experimental.pallas import tpu_sc as plsc

D = 128
CH_BIG = 128
NRING = 4
QSLOT = 8
QDEPTH = 5


@functools.cache
def _build_doc(n_rows, doc_w):
    info = plsc.get_sparse_core_info()
    nc, ns = info.num_cores, info.num_subcores
    nw = nc * ns
    rpw = n_rows // nw
    doc_rg = NRING
    doc_ng = rpw // doc_rg
    gpp = 2
    rps = gpp * doc_rg
    npiece = doc_ng // gpp
    ch_sm = doc_w - CH_BIG
    mesh = plsc.VectorSubcoreMesh(core_axis_name="c", subcore_axis_name="s")

    @functools.partial(
        pl.kernel,
        out_type=jax.ShapeDtypeStruct((n_rows, doc_w, D), jnp.float32),
        mesh=mesh,
        scratch_types=[
            pltpu.VMEM((2, rps, doc_w), jnp.int32),
            pltpu.VMEM((NRING, CH_BIG, D), jnp.float32),
            pltpu.VMEM((NRING, ch_sm, D), jnp.float32),
            pltpu.SemaphoreType.DMA((2 * NRING,)),
            pltpu.SemaphoreType.DMA((2 * NRING,)),
            pltpu.SemaphoreType.DMA,
        ],
    )
    def k(table, doc_idx, doc_out, didx_v, big_v, sm_v, gsem, osem, ssem):
        wid = lax.axis_index("s") * nc + lax.axis_index("c")
        row0 = wid * rpw

        slots = []
        for i in range(doc_rg):
            slots.append((i, 0, CH_BIG, big_v, i, i))
            slots.append((i, CH_BIG, doc_w - CH_BIG, sm_v, i, NRING + i))

        def stage_doc(p, h):
            pltpu.async_copy(
                doc_idx.at[pl.ds(row0 + p * rps, rps)], didx_v.at[h], ssem)

        def wait_stage():
            pltpu.make_async_copy(
                doc_idx.at[pl.ds(row0, rps)], didx_v.at[0], ssem).wait()

        def gather(g, slot):
            i, c, n, buf, bi, si = slot
            h = lax.rem(lax.div(g, gpp), 2)
            lr = lax.rem(g, gpp) * doc_rg + i
            pltpu.async_copy(
                table.at[didx_v.at[h, lr, pl.ds(c, n)]],
                buf.at[bi, pl.ds(0, n)], gsem.at[si])

        def wait_gather(slot):
            _, c, n, buf, bi, si = slot
            pltpu.make_async_copy(
                table.at[pl.ds(0, n)], buf.at[bi, pl.ds(0, n)],
                gsem.at[si]).wait()

        def put(g, slot):
            i, c, n, buf, bi, si = slot
            pltpu.async_copy(
                buf.at[bi, pl.ds(0, n)],
                doc_out.at[row0 + g * doc_rg + i, pl.ds(c, n)], osem.at[si])

        def wait_put(slot):
            _, c, n, buf, bi, si = slot
            pltpu.make_async_copy(
                buf.at[bi, pl.ds(0, n)], doc_out.at[0, pl.ds(c, n)],
                osem.at[si]).wait()

        stage_doc(0, 0)
        wait_stage()
        for slot in slots:
            gather(0, slot)
        stage_doc(1, 1)

        def body(g, carry):
            for slot in slots:
                wait_gather(slot)
                put(g, slot)

            @pl.when(
                jnp.logical_and(lax.rem(g, gpp) == gpp - 1,
                                g + 1 < doc_ng))
            def _():
                wait_stage()

            @pl.when(g + 1 < doc_ng)
            def _():
                for slot in slots:
                    wait_put(slot)
                    gather(g + 1, slot)

            @pl.when(g + 1 == doc_ng)
            def _():
                for slot in slots:
                    wait_put(slot)

            @pl.when(
                jnp.logical_and(lax.rem(g, gpp) == gpp - 1,
                                lax.div(g, gpp) + 2 < npiece))
            def _():
                p_next = lax.div(g, gpp) + 2
                stage_doc(p_next, lax.rem(p_next, 2))

            return carry

        lax.fori_loop(0, doc_ng, body, 0)

    return k


@functools.cache
def _build_qry(n_rows, qry_w):
    info = plsc.get_sparse_core_info()
    nc, ns = info.num_cores, info.num_subcores
    nw = nc * ns
    rpw = n_rows // nw
    mesh = plsc.VectorSubcoreMesh(core_axis_name="c", subcore_axis_name="s")

    @functools.partial(
        pl.kernel,
        out_type=jax.ShapeDtypeStruct((n_rows, qry_w, D), jnp.float32),
        mesh=mesh,
        scratch_types=[
            pltpu.VMEM((rpw, qry_w), jnp.int32),
            pltpu.VMEM((QSLOT, qry_w, D), jnp.float32),
            pltpu.SemaphoreType.DMA((QSLOT,)),
            pltpu.SemaphoreType.DMA((QSLOT,)),
        ],
    )
    def k(table, qry_idx, qry_out, qidx_v, bufs, gsem, osem):
        wid = lax.axis_index("s") * nc + lax.axis_index("c")
        row0 = wid * rpw
        pltpu.sync_copy(qry_idx.at[pl.ds(row0, rpw)], qidx_v)

        def gather(j):
            pltpu.async_copy(
                table.at[qidx_v.at[j]], bufs.at[lax.rem(j, QSLOT)],
                gsem.at[lax.rem(j, QSLOT)])

        def wait_gather(s):
            pltpu.make_async_copy(
                qry_out.at[0], bufs.at[s], gsem.at[s]).wait()

        def put(j, s):
            pltpu.async_copy(bufs.at[s], qry_out.at[row0 + j], osem.at[s])

        def wait_put(s):
            pltpu.make_async_copy(
                bufs.at[s], qry_out.at[0], osem.at[s]).wait()

        for j in range(QDEPTH):
            gather(j)

        def body(j, carry):
            s = lax.rem(j, QSLOT)
            wait_gather(s)
            put(j, s)

            @pl.when(j >= QSLOT - QDEPTH)
            def _():
                wait_put(lax.rem(j + QDEPTH, QSLOT))

            @pl.when(j + QDEPTH < rpw)
            def _():
                gather(j + QDEPTH)

            return carry

        lax.fori_loop(0, rpw, body, 0)
        for t in range(QSLOT - QDEPTH):
            wait_put((rpw - (QSLOT - QDEPTH) + t) % QSLOT)

    return k


def kernel(doc, qry, table):
    qk = _build_qry(qry.shape[0], qry.shape[1])
    dk = _build_doc(doc.shape[0], doc.shape[1])
    qry_out = qk(table, qry)
    doc_out = dk(table, doc)
    return (doc_out, qry_out)


# speedup vs baseline: 1.0026x; 1.0026x over previous
import functools

import jax
import jax.numpy as jnp
from jax import lax
from jax.experimental import pallas as pl
from jax.experimental.pallas import tpu as pltpu
from jax.experimental.pallas import tpu_sc as plsc

D = 128
CH_BIG = 128
NRING = 4
QSLOT = 8
QDEPTH = 7


@functools.cache
def _build_doc(n_rows, doc_w):
    info = plsc.get_sparse_core_info()
    nc, ns = info.num_cores, info.num_subcores
    nw = nc * ns
    rpw = n_rows // nw
    doc_rg = NRING
    doc_ng = rpw // doc_rg
    gpp = 2
    rps = gpp * doc_rg
    npiece = doc_ng // gpp
    ch_sm = doc_w - CH_BIG
    mesh = plsc.VectorSubcoreMesh(core_axis_name="c", subcore_axis_name="s")

    @functools.partial(
        pl.kernel,
        out_type=jax.ShapeDtypeStruct((n_rows, doc_w, D), jnp.float32),
        mesh=mesh,
        scratch_types=[
            pltpu.VMEM((2, rps, doc_w), jnp.int32),
            pltpu.VMEM((NRING, CH_BIG, D), jnp.float32),
            pltpu.VMEM((NRING, ch_sm, D), jnp.float32),
            pltpu.SemaphoreType.DMA((2 * NRING,)),
            pltpu.SemaphoreType.DMA((2 * NRING,)),
            pltpu.SemaphoreType.DMA,
        ],
    )
    def k(table, doc_idx, doc_out, didx_v, big_v, sm_v, gsem, osem, ssem):
        wid = lax.axis_index("s") * nc + lax.axis_index("c")
        row0 = wid * rpw

        slots = []
        for i in range(doc_rg):
            slots.append((i, 0, CH_BIG, big_v, i, i))
            slots.append((i, CH_BIG, doc_w - CH_BIG, sm_v, i, NRING + i))

        def stage_doc(p, h):
            pltpu.async_copy(
                doc_idx.at[pl.ds(row0 + p * rps, rps)], didx_v.at[h], ssem)

        def wait_stage():
            pltpu.make_async_copy(
                doc_idx.at[pl.ds(row0, rps)], didx_v.at[0], ssem).wait()

        def gather(g, slot):
            i, c, n, buf, bi, si = slot
            h = lax.rem(lax.div(g, gpp), 2)
            lr = lax.rem(g, gpp) * doc_rg + i
            pltpu.async_copy(
                table.at[didx_v.at[h, lr, pl.ds(c, n)]],
                buf.at[bi, pl.ds(0, n)], gsem.at[si])

        def wait_gather(slot):
            _, c, n, buf, bi, si = slot
            pltpu.make_async_copy(
                table.at[pl.ds(0, n)], buf.at[bi, pl.ds(0, n)],
                gsem.at[si]).wait()

        def put(g, slot):
            i, c, n, buf, bi, si = slot
            pltpu.async_copy(
                buf.at[bi, pl.ds(0, n)],
                doc_out.at[row0 + g * doc_rg + i, pl.ds(c, n)], osem.at[si])

        def wait_put(slot):
            _, c, n, buf, bi, si = slot
            pltpu.make_async_copy(
                buf.at[bi, pl.ds(0, n)], doc_out.at[0, pl.ds(c, n)],
                osem.at[si]).wait()

        stage_doc(0, 0)
        wait_stage()
        for slot in slots:
            gather(0, slot)
        stage_doc(1, 1)

        def body(g, carry):
            for slot in slots:
                wait_gather(slot)
                put(g, slot)

            @pl.when(
                jnp.logical_and(lax.rem(g, gpp) == gpp - 1,
                                g + 1 < doc_ng))
            def _():
                wait_stage()

            @pl.when(g + 1 < doc_ng)
            def _():
                for slot in slots:
                    wait_put(slot)
                    gather(g + 1, slot)

            @pl.when(g + 1 == doc_ng)
            def _():
                for slot in slots:
                    wait_put(slot)

            @pl.when(
                jnp.logical_and(lax.rem(g, gpp) == gpp - 1,
                                lax.div(g, gpp) + 2 < npiece))
            def _():
                p_next = lax.div(g, gpp) + 2
                stage_doc(p_next, lax.rem(p_next, 2))

            return carry

        lax.fori_loop(0, doc_ng, body, 0)

    return k


@functools.cache
def _build_qry(n_rows, qry_w):
    info = plsc.get_sparse_core_info()
    nc, ns = info.num_cores, info.num_subcores
    nw = nc * ns
    rpw = n_rows // nw
    mesh = plsc.VectorSubcoreMesh(core_axis_name="c", subcore_axis_name="s")

    @functools.partial(
        pl.kernel,
        out_type=jax.ShapeDtypeStruct((n_rows, qry_w, D), jnp.float32),
        mesh=mesh,
        scratch_types=[
            pltpu.VMEM((rpw, qry_w), jnp.int32),
            pltpu.VMEM((QSLOT, qry_w, D), jnp.float32),
            pltpu.SemaphoreType.DMA((QSLOT,)),
            pltpu.SemaphoreType.DMA((QSLOT,)),
        ],
    )
    def k(table, qry_idx, qry_out, qidx_v, bufs, gsem, osem):
        wid = lax.axis_index("s") * nc + lax.axis_index("c")
        row0 = wid * rpw
        pltpu.sync_copy(qry_idx.at[pl.ds(row0, rpw)], qidx_v)

        def gather(j):
            pltpu.async_copy(
                table.at[qidx_v.at[j]], bufs.at[lax.rem(j, QSLOT)],
                gsem.at[lax.rem(j, QSLOT)])

        def wait_gather(s):
            pltpu.make_async_copy(
                qry_out.at[0], bufs.at[s], gsem.at[s]).wait()

        def put(j, s):
            pltpu.async_copy(bufs.at[s], qry_out.at[row0 + j], osem.at[s])

        def wait_put(s):
            pltpu.make_async_copy(
                bufs.at[s], qry_out.at[0], osem.at[s]).wait()

        for j in range(QDEPTH):
            gather(j)

        def body(j, carry):
            s = lax.rem(j, QSLOT)
            wait_gather(s)
            put(j, s)

            @pl.when(j >= QSLOT - QDEPTH)
            def _():
                wait_put(lax.rem(j + QDEPTH, QSLOT))

            @pl.when(j + QDEPTH < rpw)
            def _():
                gather(j + QDEPTH)

            return carry

        lax.fori_loop(0, rpw, body, 0)
        for t in range(QSLOT - QDEPTH):
            wait_put((rpw - (QSLOT - QDEPTH) + t) % QSLOT)

    return k


def kernel(doc, qry, table):
    qk = _build_qry(qry.shape[0], qry.shape[1])
    dk = _build_doc(doc.shape[0], doc.shape[1])
    qry_out = qk(table, qry)
    doc_out = dk(table, doc)
    return (doc_out, qry_out)
